# baseline (speedup 1.0000x reference)
import functools

import jax
import jax.numpy as jnp
from jax import lax
from jax.experimental import pallas as pl
from jax.experimental.pallas import tpu as pltpu
from jax.experimental.pallas import tpu_sc as plsc

F32 = jnp.float32

_NC = 2
_NS = 16
_NW = _NC * _NS

_C = 100
_GSLAB = 2
_SSLAB = 2

_W = 128
_HID = 64
_PW = 16


def _silu(x):
    return x * jax.nn.sigmoid(x)


def _pack_tables(h, p, wa, wb, be1r):
    u = jnp.dot(h, wa[...], preferred_element_type=F32)
    v = jnp.dot(h, wb[...], preferred_element_type=F32) + be1r[...]
    zpad = jnp.zeros((h.shape[0], _W - _HID - _PW), F32)
    tu = jnp.concatenate([u, p, zpad], axis=1)
    tv = jnp.concatenate([v, -p, zpad], axis=1)
    return tu, tv


def _init_body(xin, p, win, bin_, wa, wb, be1r, h_ref, tu_ref, tv_ref):
    h = jnp.dot(xin[...], win[...], preferred_element_type=F32) + bin_[...]
    h_ref[...] = h
    tu_ref[...], tv_ref[...] = _pack_tables(h, p[...], wa, wb, be1r)


def _edge_body(gs, gd, wc, we2, be2r, wxr, bxr, mw_ref):
    s = gs[...] + gd[...]
    uv = s[:, :_HID]
    rel = s[:, _HID:_HID + _PW]
    d2 = jnp.sum(rel * rel, axis=1, keepdims=True)
    m1 = _silu(uv + d2 * wc[...])
    m = _silu(jnp.dot(m1, we2[...], preferred_element_type=F32) + be2r[...])
    w = jnp.tanh(jnp.sum(m * wxr[...], axis=1, keepdims=True) + bxr[0, 0])
    zpad = jnp.zeros((m.shape[0], _W - _HID - _PW), F32)
    mw_ref[...] = jnp.concatenate([m, rel * w, zpad], axis=1)


def _node_body(h, p, pm0, pm1, pm2, pm3, wh1a, wh1b, bh1r, wh2, bh2r, wa, wb,
               be1r, hn_ref, pn_ref, tu_ref, tv_ref, *, inv_deg):
    acc = ((pm0[...] + pm1[...]) + (pm2[...] + pm3[...])) * inv_deg
    agg = acc[:, :_HID]
    dp = acc[:, _HID:_HID + _PW]
    pn = p[...] + dp
    pn_ref[...] = pn
    t1 = _silu(jnp.dot(h[...], wh1a[...], preferred_element_type=F32)
               + jnp.dot(agg, wh1b[...], preferred_element_type=F32)
               + bh1r[...])
    hn = h[...] + _silu(jnp.dot(t1, wh2[...], preferred_element_type=F32)
                        + bh2r[...])
    hn_ref[...] = hn
    tu_ref[...], tv_ref[...] = _pack_tables(hn, pn, wa, wb, be1r)


def _node_final_body(h, pm0, pm1, pm2, pm3, wh1a, wh1b, bh1r, wh2, bh2r,
                     wo1, bo1r, wo2, bo2r, o_ref, *, inv_deg):
    acc = ((pm0[...] + pm1[...]) + (pm2[...] + pm3[...])) * inv_deg
    agg = acc[:, :_HID]
    t1 = _silu(jnp.dot(h[...], wh1a[...], preferred_element_type=F32)
               + jnp.dot(agg, wh1b[...], preferred_element_type=F32)
               + bh1r[...])
    hn = h[...] + _silu(jnp.dot(t1, wh2[...], preferred_element_type=F32)
                        + bh2r[...])
    t2 = _silu(jnp.dot(hn, wo1[...], preferred_element_type=F32) + bo1r[...])
    o_ref[...] = jnp.dot(t2, wo2[...], preferred_element_type=F32) + bo2r[...]


def _make_gather(e):
    ew = e // _NW
    cs = _C * _GSLAB
    nslab = ew // cs
    nch = nslab * _GSLAB
    mesh = plsc.VectorSubcoreMesh(core_axis_name="c", subcore_axis_name="s")

    @functools.partial(
        pl.kernel,
        out_type=(
            jax.ShapeDtypeStruct((e, _W), F32),
            jax.ShapeDtypeStruct((e, _W), F32),
        ),
        mesh=mesh,
        scratch_types=(
            pltpu.VMEM((nch, _C), jnp.int32),
            pltpu.VMEM((nch, _C), jnp.int32),
            pltpu.VMEM((cs, _W), F32),
            pltpu.VMEM((cs, _W), F32),
            pltpu.VMEM((cs, _W), F32),
            pltpu.VMEM((cs, _W), F32),
            pltpu.SemaphoreType.DMA,
            pltpu.SemaphoreType.DMA,
            pltpu.SemaphoreType.DMA,
        ),
    )
    def gather_k(tu_h, tv_h, src_h, dst_h, gs_h, gd_h,
                 si, di, ub_a, vb_a, ub_b, vb_b, sg, sw_a, sw_b):
        wid = lax.axis_index("s") * _NC + lax.axis_index("c")
        i1 = pltpu.async_copy(src_h.at[wid], si, sg)
        i2 = pltpu.async_copy(dst_h.at[wid], di, sg)
        i1.wait()
        i2.wait()

        def phase(j, s, ub, vb, sw):
            @pl.when(j > 0)
            def _():
                pltpu.make_async_copy(ub, gs_h.at[pl.ds(0, cs)], sw).wait()
                pltpu.make_async_copy(vb, gd_h.at[pl.ds(0, cs)], sw).wait()

            cps = []
            for k in range(_GSLAB):
                ch = s * _GSLAB + k
                o = k * _C
                cps.append(pltpu.async_copy(tu_h.at[si.at[ch]],
                                            ub.at[pl.ds(o, _C)], sg))
                cps.append(pltpu.async_copy(tv_h.at[di.at[ch]],
                                            vb.at[pl.ds(o, _C)], sg))
            for cp in cps:
                cp.wait()
            rb = wid * ew + s * cs
            pltpu.async_copy(ub, gs_h.at[pl.ds(rb, cs)], sw)
            pltpu.async_copy(vb, gd_h.at[pl.ds(rb, cs)], sw)

        def body(j, carry):
            phase(j, 2 * j, ub_a, vb_a, sw_a)
            phase(j, 2 * j + 1, ub_b, vb_b, sw_b)
            return carry

        half = nslab // 2
        lax.fori_loop(0, half, body, 0)
        if nslab % 2:
            phase(half, nslab - 1, ub_a, vb_a, sw_a)
        pltpu.make_async_copy(ub_a, gs_h.at[pl.ds(0, cs)], sw_a).wait()
        pltpu.make_async_copy(vb_a, gd_h.at[pl.ds(0, cs)], sw_a).wait()
        pltpu.make_async_copy(ub_b, gs_h.at[pl.ds(0, cs)], sw_b).wait()
        pltpu.make_async_copy(vb_b, gd_h.at[pl.ds(0, cs)], sw_b).wait()

    return gather_k


def _make_scatter(n, e):
    ew = e // _NW
    cs = _C * _SSLAB
    nslab = ew // cs
    stride = 624
    span = n - (_NS - 1) * stride
    mesh = plsc.VectorSubcoreMesh(core_axis_name="c", subcore_axis_name="s")

    @functools.partial(
        pl.kernel,
        out_type=(
            jax.ShapeDtypeStruct((n, _W), F32),
            jax.ShapeDtypeStruct((n, _W), F32),
        ),
        mesh=mesh,
        scratch_types=(
            pltpu.VMEM((nslab * _SSLAB, _C), jnp.int32),
            pltpu.VMEM((cs, _W), F32),
            pltpu.VMEM_SHARED((n, _W), F32),
            pltpu.SemaphoreType.DMA,
            pltpu.SemaphoreType.DMA,
        ),
    )
    def scatter_k(mw_h, dst_h, z_h, pm0_h, pm1_h, di, mb, am, sl, ss):
        cid = lax.axis_index("c")
        sid = lax.axis_index("s")
        wid = sid * _NC + cid
        r0 = sid * stride
        l0 = pltpu.async_copy(dst_h.at[wid], di, sl)
        pltpu.sync_copy(z_h.at[pl.ds(r0, span)], am.at[pl.ds(r0, span)])
        l0.wait()
        plsc.subcore_barrier()

        def slab(s, carry):
            base = wid * ew + s * cs
            l1 = pltpu.async_copy(mw_h.at[pl.ds(base, cs)], mb, sl)
            l1.wait()
            cps = []
            for k in range(_SSLAB):
                ch = s * _SSLAB + k
                cps.append(pltpu.async_copy(mb.at[pl.ds(k * _C, _C)],
                                            am.at[di.at[ch]], ss, add=True))
            for cp in cps:
                cp.wait()
            return carry

        lax.fori_loop(0, nslab, slab, 0)
        plsc.subcore_barrier()

        @pl.when(cid == 0)
        def _():
            pltpu.sync_copy(am.at[pl.ds(r0, span)], pm0_h.at[pl.ds(r0, span)])

        @pl.when(cid == 1)
        def _():
            pltpu.sync_copy(am.at[pl.ds(r0, span)], pm1_h.at[pl.ds(r0, span)])

    return scatter_k


def kernel(feats, pos, edge_index, x_t, t, T, W_in, b_in, We1, be1, We2, be2,
           Wx, bx, Wh1, bh1, Wh2, bh2, Wo1, bo1, Wo2, bo2):
    b, l_, f = feats.shape
    n = b * l_
    td = x_t.shape[-1]
    e = edge_index.shape[1]
    nl = We1.shape[0]
    hid = W_in.shape[1]
    inv_deg = float(n) / float(e)

    bn = 2000
    be_blk = 6400
    nb = n // bn

    t_norm = jnp.clip(t.astype(F32) / jnp.asarray(T).astype(F32), 0.0, 1.0)
    t_feat = jnp.broadcast_to(t_norm[:, None, None], (b, l_, 1))
    x_in = jnp.concatenate([feats, x_t, t_feat], axis=-1).reshape(n, -1)
    in_dim = x_in.shape[1]
    p4 = jnp.pad(pos.reshape(n, 3).astype(F32), ((0, 0), (0, _PW - 3)))
    e_a = 2 * e // 5
    sizes = [e_a, e - e_a]
    offs = [0, e_a]

    def _idx(rowi, sz, off, chunk):
        return edge_index[rowi, off:off + sz].reshape(
            _NW, (sz // _NW) // (_C * chunk) * chunk, _C)

    src_g = [_idx(0, sz, o, _GSLAB) for sz, o in zip(sizes, offs)]
    dst_g = [_idx(1, sz, o, _GSLAB) for sz, o in zip(sizes, offs)]
    dst_s = [_idx(1, sz, o, _SSLAB) for sz, o in zip(sizes, offs)]
    zmw = jnp.zeros((n, _W), F32)
    wo2p = jnp.pad(Wo2, ((0, 0), (0, _W - td)))
    bo2p = jnp.pad(bo2, (0, _W - td))

    row = lambda a: a.reshape(1, -1)

    wfull = lambda s: pl.BlockSpec(s, lambda i: (0, 0))
    nblk = lambda w: pl.BlockSpec((bn, w), lambda i: (i, 0))
    eblk = lambda w: pl.BlockSpec((be_blk, w), lambda i: (i, 0))

    h, tu, tv = pl.pallas_call(
        _init_body,
        grid=(nb,),
        in_specs=[
            nblk(in_dim), nblk(_PW), wfull((in_dim, hid)), wfull((1, hid)),
            wfull((hid, hid)), wfull((hid, hid)), wfull((1, hid)),
        ],
        out_specs=[nblk(hid), nblk(_W), nblk(_W)],
        out_shape=[
            jax.ShapeDtypeStruct((n, hid), F32),
            jax.ShapeDtypeStruct((n, _W), F32),
            jax.ShapeDtypeStruct((n, _W), F32),
        ],
    )(x_in, p4, W_in, row(b_in), We1[0, :hid], We1[0, hid:2 * hid],
      row(be1[0]))

    gather_ks = [_make_gather(sz) for sz in sizes]
    scatter_ks = [_make_scatter(n, sz) for sz in sizes]

    def edge_mlp(gs, gd, l, sz):
        return pl.pallas_call(
            _edge_body,
            grid=(sz // be_blk,),
            in_specs=[
                eblk(_W), eblk(_W),
                wfull((1, hid)), wfull((hid, hid)), wfull((1, hid)),
                wfull((1, hid)),
                pl.BlockSpec(memory_space=pltpu.SMEM),
            ],
            out_specs=eblk(_W),
            out_shape=jax.ShapeDtypeStruct((sz, _W), F32),
        )(gs, gd, row(We1[l, 2 * hid]), We2[l], row(be2[l]),
          row(Wx[l, :, 0]), bx[l].reshape(1, 1))

    p = p4
    for l in range(nl):
        gs_a, gd_a = gather_ks[0](tu, tv, src_g[0], dst_g[0])
        gs_b, gd_b = gather_ks[1](tu, tv, src_g[1], dst_g[1])
        mw_a = edge_mlp(gs_a, gd_a, l, sizes[0])
        mw_b = edge_mlp(gs_b, gd_b, l, sizes[1])
        pm0, pm1 = scatter_ks[0](mw_a, dst_s[0], zmw)
        pm2, pm3 = scatter_ks[1](mw_b, dst_s[1], zmw)

        if l + 1 < nl:
            la = l + 1
            h, p, tu, tv = pl.pallas_call(
                functools.partial(_node_body, inv_deg=inv_deg),
                grid=(nb,),
                in_specs=[
                    nblk(hid), nblk(_PW), nblk(_W), nblk(_W), nblk(_W),
                    nblk(_W),
                    wfull((hid, hid)), wfull((hid, hid)), wfull((1, hid)),
                    wfull((hid, hid)), wfull((1, hid)),
                    wfull((hid, hid)), wfull((hid, hid)), wfull((1, hid)),
                ],
                out_specs=[nblk(hid), nblk(_PW), nblk(_W), nblk(_W)],
                out_shape=[
                    jax.ShapeDtypeStruct((n, hid), F32),
                    jax.ShapeDtypeStruct((n, _PW), F32),
                    jax.ShapeDtypeStruct((n, _W), F32),
                    jax.ShapeDtypeStruct((n, _W), F32),
                ],
            )(h, p, pm0, pm1, pm2, pm3, Wh1[l, :hid], Wh1[l, hid:],
              row(bh1[l]), Wh2[l], row(bh2[l]), We1[la, :hid],
              We1[la, hid:2 * hid], row(be1[la]))
        else:
            pred = pl.pallas_call(
                functools.partial(_node_final_body, inv_deg=inv_deg),
                grid=(nb,),
                in_specs=[
                    nblk(hid), nblk(_W), nblk(_W), nblk(_W), nblk(_W),
                    wfull((hid, hid)), wfull((hid, hid)), wfull((1, hid)),
                    wfull((hid, hid)), wfull((1, hid)),
                    wfull((hid, hid)), wfull((1, hid)),
                    wfull((hid, _W)), wfull((1, _W)),
                ],
                out_specs=nblk(_W),
                out_shape=jax.ShapeDtypeStruct((n, _W), F32),
            )(h, pm0, pm1, pm2, pm3, Wh1[l, :hid], Wh1[l, hid:],
              row(bh1[l]), Wh2[l], row(bh2[l]), Wo1, row(bo1), wo2p,
              bo2p.reshape(1, -1))

    return pred[:, :td].reshape(b, l_, td)


# speedup vs baseline: 1.0248x; 1.0248x over previous
import functools

import jax
import jax.numpy as jnp
from jax import lax
from jax.experimental import pallas as pl
from jax.experimental.pallas import tpu as pltpu
from jax.experimental.pallas import tpu_sc as plsc

F32 = jnp.float32

_NC = 2
_NS = 16
_NW = _NC * _NS

_C = 100
_GSLAB = 2
_SSLAB = 2

_W = 128
_HID = 64
_PW = 16


def _silu(x):
    return x * jax.nn.sigmoid(x)


def _pack_tables(h, p, wa, wb, be1r):
    u = jnp.dot(h, wa[...], preferred_element_type=F32)
    v = jnp.dot(h, wb[...], preferred_element_type=F32) + be1r[...]
    zpad = jnp.zeros((h.shape[0], _W - _HID - _PW), F32)
    tu = jnp.concatenate([u, p, zpad], axis=1)
    tv = jnp.concatenate([v, -p, zpad], axis=1)
    return tu, tv


def _init_body(xin, p, win, bin_, wa, wb, be1r, h_ref, tu_ref, tv_ref):
    h = jnp.dot(xin[...], win[...], preferred_element_type=F32) + bin_[...]
    h_ref[...] = h
    tu_ref[...], tv_ref[...] = _pack_tables(h, p[...], wa, wb, be1r)


def _edge_body(gs, gd, wc, we2, be2r, wxr, bxr, mw_ref):
    s = gs[...] + gd[...]
    uv = s[:, :_HID]
    rel = s[:, _HID:_HID + _PW]
    d2 = jnp.sum(rel * rel, axis=1, keepdims=True)
    m1 = _silu(uv + d2 * wc[...])
    m = _silu(jnp.dot(m1, we2[...], preferred_element_type=F32) + be2r[...])
    w = jnp.tanh(jnp.sum(m * wxr[...], axis=1, keepdims=True) + bxr[0, 0])
    zpad = jnp.zeros((m.shape[0], _W - _HID - _PW), F32)
    mw_ref[...] = jnp.concatenate([m, rel * w, zpad], axis=1)


def _node_body(h, p, pm0, pm1, pm2, pm3, wh1a, wh1b, bh1r, wh2, bh2r, wa, wb,
               be1r, hn_ref, pn_ref, tu_ref, tv_ref, *, inv_deg):
    acc = ((pm0[...] + pm1[...]) + (pm2[...] + pm3[...])) * inv_deg
    agg = acc[:, :_HID]
    dp = acc[:, _HID:_HID + _PW]
    pn = p[...] + dp
    pn_ref[...] = pn
    t1 = _silu(jnp.dot(h[...], wh1a[...], preferred_element_type=F32)
               + jnp.dot(agg, wh1b[...], preferred_element_type=F32)
               + bh1r[...])
    hn = h[...] + _silu(jnp.dot(t1, wh2[...], preferred_element_type=F32)
                        + bh2r[...])
    hn_ref[...] = hn
    tu_ref[...], tv_ref[...] = _pack_tables(hn, pn, wa, wb, be1r)


def _node_final_body(h, pm0, pm1, pm2, pm3, wh1a, wh1b, bh1r, wh2, bh2r,
                     wo1, bo1r, wo2, bo2r, o_ref, *, inv_deg):
    acc = ((pm0[...] + pm1[...]) + (pm2[...] + pm3[...])) * inv_deg
    agg = acc[:, :_HID]
    t1 = _silu(jnp.dot(h[...], wh1a[...], preferred_element_type=F32)
               + jnp.dot(agg, wh1b[...], preferred_element_type=F32)
               + bh1r[...])
    hn = h[...] + _silu(jnp.dot(t1, wh2[...], preferred_element_type=F32)
                        + bh2r[...])
    t2 = _silu(jnp.dot(hn, wo1[...], preferred_element_type=F32) + bo1r[...])
    o_ref[...] = jnp.dot(t2, wo2[...], preferred_element_type=F32) + bo2r[...]


def _make_gather(e):
    ew = e // _NW
    cs = _C * _GSLAB
    nslab = ew // cs
    nch = nslab * _GSLAB
    mesh = plsc.VectorSubcoreMesh(core_axis_name="c", subcore_axis_name="s")

    @functools.partial(
        pl.kernel,
        out_type=(
            jax.ShapeDtypeStruct((e, _W), F32),
            jax.ShapeDtypeStruct((e, _W), F32),
        ),
        mesh=mesh,
        scratch_types=(
            pltpu.VMEM((nch, _C), jnp.int32),
            pltpu.VMEM((nch, _C), jnp.int32),
            pltpu.VMEM((cs, _W), F32),
            pltpu.VMEM((cs, _W), F32),
            pltpu.VMEM((cs, _W), F32),
            pltpu.VMEM((cs, _W), F32),
            pltpu.SemaphoreType.DMA,
            pltpu.SemaphoreType.DMA,
            pltpu.SemaphoreType.DMA,
        ),
    )
    def gather_k(tu_h, tv_h, src_h, dst_h, gs_h, gd_h,
                 si, di, ub_a, vb_a, ub_b, vb_b, sg, sw_a, sw_b):
        wid = lax.axis_index("s") * _NC + lax.axis_index("c")
        i1 = pltpu.async_copy(src_h.at[wid], si, sg)
        i2 = pltpu.async_copy(dst_h.at[wid], di, sg)
        i1.wait()
        i2.wait()

        def phase(j, s, ub, vb, sw):
            @pl.when(j > 0)
            def _():
                pltpu.make_async_copy(ub, gs_h.at[pl.ds(0, cs)], sw).wait()
                pltpu.make_async_copy(vb, gd_h.at[pl.ds(0, cs)], sw).wait()

            cps = []
            for k in range(_GSLAB):
                ch = s * _GSLAB + k
                o = k * _C
                cps.append(pltpu.async_copy(tu_h.at[si.at[ch]],
                                            ub.at[pl.ds(o, _C)], sg))
                cps.append(pltpu.async_copy(tv_h.at[di.at[ch]],
                                            vb.at[pl.ds(o, _C)], sg))
            for cp in cps:
                cp.wait()
            rb = wid * ew + s * cs
            pltpu.async_copy(ub, gs_h.at[pl.ds(rb, cs)], sw)
            pltpu.async_copy(vb, gd_h.at[pl.ds(rb, cs)], sw)

        def body(j, carry):
            phase(j, 2 * j, ub_a, vb_a, sw_a)
            phase(j, 2 * j + 1, ub_b, vb_b, sw_b)
            return carry

        half = nslab // 2
        lax.fori_loop(0, half, body, 0)
        if nslab % 2:
            phase(half, nslab - 1, ub_a, vb_a, sw_a)
        pltpu.make_async_copy(ub_a, gs_h.at[pl.ds(0, cs)], sw_a).wait()
        pltpu.make_async_copy(vb_a, gd_h.at[pl.ds(0, cs)], sw_a).wait()
        pltpu.make_async_copy(ub_b, gs_h.at[pl.ds(0, cs)], sw_b).wait()
        pltpu.make_async_copy(vb_b, gd_h.at[pl.ds(0, cs)], sw_b).wait()

    return gather_k


def _make_scatter(n, e):
    ew = e // _NW
    cs = _C * _SSLAB
    nslab = ew // cs
    stride = 624
    span = n - (_NS - 1) * stride
    mesh = plsc.VectorSubcoreMesh(core_axis_name="c", subcore_axis_name="s")

    @functools.partial(
        pl.kernel,
        out_type=(
            jax.ShapeDtypeStruct((n, _W), F32),
            jax.ShapeDtypeStruct((n, _W), F32),
        ),
        mesh=mesh,
        scratch_types=(
            pltpu.VMEM((nslab * _SSLAB, _C), jnp.int32),
            pltpu.VMEM((cs, _W), F32),
            pltpu.VMEM_SHARED((n, _W), F32),
            pltpu.SemaphoreType.DMA,
            pltpu.SemaphoreType.DMA,
        ),
    )
    def scatter_k(mw_h, dst_h, z_h, pm0_h, pm1_h, di, mb, am, sl, ss):
        cid = lax.axis_index("c")
        sid = lax.axis_index("s")
        wid = sid * _NC + cid
        r0 = sid * stride
        l0 = pltpu.async_copy(dst_h.at[wid], di, sl)
        pltpu.sync_copy(z_h.at[pl.ds(r0, span)], am.at[pl.ds(r0, span)])
        l0.wait()
        plsc.subcore_barrier()

        def slab(s, carry):
            base = wid * ew + s * cs
            l1 = pltpu.async_copy(mw_h.at[pl.ds(base, cs)], mb, sl)
            l1.wait()
            cps = []
            for k in range(_SSLAB):
                ch = s * _SSLAB + k
                cps.append(pltpu.async_copy(mb.at[pl.ds(k * _C, _C)],
                                            am.at[di.at[ch]], ss, add=True))
            for cp in cps:
                cp.wait()
            return carry

        lax.fori_loop(0, nslab, slab, 0)
        plsc.subcore_barrier()

        @pl.when(cid == 0)
        def _():
            pltpu.sync_copy(am.at[pl.ds(r0, span)], pm0_h.at[pl.ds(r0, span)])

        @pl.when(cid == 1)
        def _():
            pltpu.sync_copy(am.at[pl.ds(r0, span)], pm1_h.at[pl.ds(r0, span)])

    return scatter_k


def kernel(feats, pos, edge_index, x_t, t, T, W_in, b_in, We1, be1, We2, be2,
           Wx, bx, Wh1, bh1, Wh2, bh2, Wo1, bo1, Wo2, bo2):
    b, l_, f = feats.shape
    n = b * l_
    td = x_t.shape[-1]
    e = edge_index.shape[1]
    nl = We1.shape[0]
    hid = W_in.shape[1]
    inv_deg = float(n) / float(e)

    bn = 2000
    be_blk = 6400
    nb = n // bn

    t_norm = jnp.clip(t.astype(F32) / jnp.asarray(T).astype(F32), 0.0, 1.0)
    t_feat = jnp.broadcast_to(t_norm[:, None, None], (b, l_, 1))
    x_in = jnp.concatenate([feats, x_t, t_feat], axis=-1).reshape(n, -1)
    in_dim = x_in.shape[1]
    p4 = jnp.pad(pos.reshape(n, 3).astype(F32), ((0, 0), (0, _PW - 3)))
    e_a = e // 2
    sizes = [e_a, e - e_a]
    offs = [0, e_a]

    def _idx(rowi, sz, off, chunk):
        return edge_index[rowi, off:off + sz].reshape(
            _NW, (sz // _NW) // (_C * chunk) * chunk, _C)

    src_g = [_idx(0, sz, o, _GSLAB) for sz, o in zip(sizes, offs)]
    dst_g = [_idx(1, sz, o, _GSLAB) for sz, o in zip(sizes, offs)]
    dst_s = [_idx(1, sz, o, _SSLAB) for sz, o in zip(sizes, offs)]
    zmw = jnp.zeros((n, _W), F32)
    wo2p = jnp.pad(Wo2, ((0, 0), (0, _W - td)))
    bo2p = jnp.pad(bo2, (0, _W - td))

    row = lambda a: a.reshape(1, -1)

    wfull = lambda s: pl.BlockSpec(s, lambda i: (0, 0))
    nblk = lambda w: pl.BlockSpec((bn, w), lambda i: (i, 0))
    eblk = lambda w: pl.BlockSpec((be_blk, w), lambda i: (i, 0))

    h, tu, tv = pl.pallas_call(
        _init_body,
        grid=(nb,),
        in_specs=[
            nblk(in_dim), nblk(_PW), wfull((in_dim, hid)), wfull((1, hid)),
            wfull((hid, hid)), wfull((hid, hid)), wfull((1, hid)),
        ],
        out_specs=[nblk(hid), nblk(_W), nblk(_W)],
        out_shape=[
            jax.ShapeDtypeStruct((n, hid), F32),
            jax.ShapeDtypeStruct((n, _W), F32),
            jax.ShapeDtypeStruct((n, _W), F32),
        ],
    )(x_in, p4, W_in, row(b_in), We1[0, :hid], We1[0, hid:2 * hid],
      row(be1[0]))

    gather_ks = [_make_gather(sz) for sz in sizes]
    scatter_ks = [_make_scatter(n, sz) for sz in sizes]

    def edge_mlp(gs, gd, l, sz):
        return pl.pallas_call(
            _edge_body,
            grid=(sz // be_blk,),
            in_specs=[
                eblk(_W), eblk(_W),
                wfull((1, hid)), wfull((hid, hid)), wfull((1, hid)),
                wfull((1, hid)),
                pl.BlockSpec(memory_space=pltpu.SMEM),
            ],
            out_specs=eblk(_W),
            out_shape=jax.ShapeDtypeStruct((sz, _W), F32),
        )(gs, gd, row(We1[l, 2 * hid]), We2[l], row(be2[l]),
          row(Wx[l, :, 0]), bx[l].reshape(1, 1))

    p = p4
    for l in range(nl):
        gs_a, gd_a = gather_ks[0](tu, tv, src_g[0], dst_g[0])
        gs_b, gd_b = gather_ks[1](tu, tv, src_g[1], dst_g[1])
        mw_a = edge_mlp(gs_a, gd_a, l, sizes[0])
        mw_b = edge_mlp(gs_b, gd_b, l, sizes[1])
        pm0, pm1 = scatter_ks[0](mw_a, dst_s[0], zmw)
        pm2, pm3 = scatter_ks[1](mw_b, dst_s[1], zmw)

        if l + 1 < nl:
            la = l + 1
            h, p, tu, tv = pl.pallas_call(
                functools.partial(_node_body, inv_deg=inv_deg),
                grid=(nb,),
                in_specs=[
                    nblk(hid), nblk(_PW), nblk(_W), nblk(_W), nblk(_W),
                    nblk(_W),
                    wfull((hid, hid)), wfull((hid, hid)), wfull((1, hid)),
                    wfull((hid, hid)), wfull((1, hid)),
                    wfull((hid, hid)), wfull((hid, hid)), wfull((1, hid)),
                ],
                out_specs=[nblk(hid), nblk(_PW), nblk(_W), nblk(_W)],
                out_shape=[
                    jax.ShapeDtypeStruct((n, hid), F32),
                    jax.ShapeDtypeStruct((n, _PW), F32),
                    jax.ShapeDtypeStruct((n, _W), F32),
                    jax.ShapeDtypeStruct((n, _W), F32),
                ],
            )(h, p, pm0, pm1, pm2, pm3, Wh1[l, :hid], Wh1[l, hid:],
              row(bh1[l]), Wh2[l], row(bh2[l]), We1[la, :hid],
              We1[la, hid:2 * hid], row(be1[la]))
        else:
            pred = pl.pallas_call(
                functools.partial(_node_final_body, inv_deg=inv_deg),
                grid=(nb,),
                in_specs=[
                    nblk(hid), nblk(_W), nblk(_W), nblk(_W), nblk(_W),
                    wfull((hid, hid)), wfull((hid, hid)), wfull((1, hid)),
                    wfull((hid, hid)), wfull((1, hid)),
                    wfull((hid, hid)), wfull((1, hid)),
                    wfull((hid, _W)), wfull((1, _W)),
                ],
                out_specs=nblk(_W),
                out_shape=jax.ShapeDtypeStruct((n, _W), F32),
            )(h, pm0, pm1, pm2, pm3, Wh1[l, :hid], Wh1[l, hid:],
              row(bh1[l]), Wh2[l], row(bh2[l]), Wo1, row(bo1), wo2p,
              bo2p.reshape(1, -1))

    return pred[:, :td].reshape(b, l_, td)


# speedup vs baseline: 1.0293x; 1.0044x over previous
import functools

import jax
import jax.numpy as jnp
from jax import lax
from jax.experimental import pallas as pl
from jax.experimental.pallas import tpu as pltpu
from jax.experimental.pallas import tpu_sc as plsc

F32 = jnp.float32

_NC = 2
_NS = 16
_NW = _NC * _NS

_C = 100
_GSLAB = 2
_SSLAB = 2

_W = 128
_HID = 64
_PW = 16


def _silu(x):
    return x * jax.nn.sigmoid(x)


def _pack_tables(h, p, wa, wb, be1r):
    u = jnp.dot(h, wa[...], preferred_element_type=F32)
    v = jnp.dot(h, wb[...], preferred_element_type=F32) + be1r[...]
    zpad = jnp.zeros((h.shape[0], _W - _HID - _PW), F32)
    tu = jnp.concatenate([u, p, zpad], axis=1)
    tv = jnp.concatenate([v, -p, zpad], axis=1)
    return tu, tv


def _init_body(xin, p, win, bin_, wa, wb, be1r, h_ref, tu_ref, tv_ref):
    h = jnp.dot(xin[...], win[...], preferred_element_type=F32) + bin_[...]
    h_ref[...] = h
    tu_ref[...], tv_ref[...] = _pack_tables(h, p[...], wa, wb, be1r)


def _edge_body(gs, gd, wc, we2, be2r, wxr, bxr, mw_ref):
    s = gs[...] + gd[...]
    uv = s[:, :_HID]
    rel = s[:, _HID:_HID + _PW]
    d2 = jnp.sum(rel * rel, axis=1, keepdims=True)
    m1 = _silu(uv + d2 * wc[...])
    m = _silu(jnp.dot(m1, we2[...], preferred_element_type=F32) + be2r[...])
    w = jnp.tanh(jnp.sum(m * wxr[...], axis=1, keepdims=True) + bxr[0, 0])
    zpad = jnp.zeros((m.shape[0], _W - _HID - _PW), F32)
    mw_ref[...] = jnp.concatenate([m, rel * w, zpad], axis=1)


def _node_body(h, p, pm0, pm1, pm2, pm3, wh1a, wh1b, bh1r, wh2, bh2r, wa, wb,
               be1r, hn_ref, pn_ref, tu_ref, tv_ref, *, inv_deg):
    acc = ((pm0[...] + pm1[...]) + (pm2[...] + pm3[...])) * inv_deg
    agg = acc[:, :_HID]
    dp = acc[:, _HID:_HID + _PW]
    pn = p[...] + dp
    pn_ref[...] = pn
    t1 = _silu(jnp.dot(h[...], wh1a[...], preferred_element_type=F32)
               + jnp.dot(agg, wh1b[...], preferred_element_type=F32)
               + bh1r[...])
    hn = h[...] + _silu(jnp.dot(t1, wh2[...], preferred_element_type=F32)
                        + bh2r[...])
    hn_ref[...] = hn
    tu_ref[...], tv_ref[...] = _pack_tables(hn, pn, wa, wb, be1r)


def _node_final_body(h, pm0, pm1, pm2, pm3, wh1a, wh1b, bh1r, wh2, bh2r,
                     wo1, bo1r, wo2, bo2r, o_ref, *, inv_deg):
    acc = ((pm0[...] + pm1[...]) + (pm2[...] + pm3[...])) * inv_deg
    agg = acc[:, :_HID]
    t1 = _silu(jnp.dot(h[...], wh1a[...], preferred_element_type=F32)
               + jnp.dot(agg, wh1b[...], preferred_element_type=F32)
               + bh1r[...])
    hn = h[...] + _silu(jnp.dot(t1, wh2[...], preferred_element_type=F32)
                        + bh2r[...])
    t2 = _silu(jnp.dot(hn, wo1[...], preferred_element_type=F32) + bo1r[...])
    o_ref[...] = jnp.dot(t2, wo2[...], preferred_element_type=F32) + bo2r[...]


def _make_gather(e):
    ew = e // _NW
    cs = _C * _GSLAB
    nslab = ew // cs
    nch = nslab * _GSLAB
    mesh = plsc.VectorSubcoreMesh(core_axis_name="c", subcore_axis_name="s")

    @functools.partial(
        pl.kernel,
        out_type=(
            jax.ShapeDtypeStruct((e, _W), F32),
            jax.ShapeDtypeStruct((e, _W), F32),
        ),
        mesh=mesh,
        scratch_types=(
            pltpu.VMEM((nch, _C), jnp.int32),
            pltpu.VMEM((nch, _C), jnp.int32),
            pltpu.VMEM((cs, _W), F32),
            pltpu.VMEM((cs, _W), F32),
            pltpu.VMEM((cs, _W), F32),
            pltpu.VMEM((cs, _W), F32),
            pltpu.SemaphoreType.DMA,
            pltpu.SemaphoreType.DMA,
            pltpu.SemaphoreType.DMA,
        ),
    )
    def gather_k(tu_h, tv_h, src_h, dst_h, gs_h, gd_h,
                 si, di, ub_a, vb_a, ub_b, vb_b, sg, sw_a, sw_b):
        wid = lax.axis_index("s") * _NC + lax.axis_index("c")
        i1 = pltpu.async_copy(src_h.at[wid], si, sg)
        i2 = pltpu.async_copy(dst_h.at[wid], di, sg)
        i1.wait()
        i2.wait()

        def phase(j, s, ub, vb, sw):
            @pl.when(j > 0)
            def _():
                pltpu.make_async_copy(ub, gs_h.at[pl.ds(0, cs)], sw).wait()
                pltpu.make_async_copy(vb, gd_h.at[pl.ds(0, cs)], sw).wait()

            cps = []
            for k in range(_GSLAB):
                ch = s * _GSLAB + k
                o = k * _C
                cps.append(pltpu.async_copy(tu_h.at[si.at[ch]],
                                            ub.at[pl.ds(o, _C)], sg))
                cps.append(pltpu.async_copy(tv_h.at[di.at[ch]],
                                            vb.at[pl.ds(o, _C)], sg))
            for cp in cps:
                cp.wait()
            rb = wid * ew + s * cs
            pltpu.async_copy(ub, gs_h.at[pl.ds(rb, cs)], sw)
            pltpu.async_copy(vb, gd_h.at[pl.ds(rb, cs)], sw)

        def body(j, carry):
            phase(j, 2 * j, ub_a, vb_a, sw_a)
            phase(j, 2 * j + 1, ub_b, vb_b, sw_b)
            return carry

        half = nslab // 2
        lax.fori_loop(0, half, body, 0)
        if nslab % 2:
            phase(half, nslab - 1, ub_a, vb_a, sw_a)
        pltpu.make_async_copy(ub_a, gs_h.at[pl.ds(0, cs)], sw_a).wait()
        pltpu.make_async_copy(vb_a, gd_h.at[pl.ds(0, cs)], sw_a).wait()
        pltpu.make_async_copy(ub_b, gs_h.at[pl.ds(0, cs)], sw_b).wait()
        pltpu.make_async_copy(vb_b, gd_h.at[pl.ds(0, cs)], sw_b).wait()

    return gather_k


def _make_scatter(n, e):
    ew = e // _NW
    cs = _C * _SSLAB
    nslab = ew // cs
    stride = 624
    span = n - (_NS - 1) * stride
    mesh = plsc.VectorSubcoreMesh(core_axis_name="c", subcore_axis_name="s")

    @functools.partial(
        pl.kernel,
        out_type=(
            jax.ShapeDtypeStruct((n, _W), F32),
            jax.ShapeDtypeStruct((n, _W), F32),
        ),
        mesh=mesh,
        scratch_types=(
            pltpu.VMEM((nslab * _SSLAB, _C), jnp.int32),
            pltpu.VMEM((cs, _W), F32),
            pltpu.VMEM_SHARED((n, _W), F32),
            pltpu.SemaphoreType.DMA,
            pltpu.SemaphoreType.DMA,
        ),
    )
    def scatter_k(mw_h, dst_h, z_h, pm0_h, pm1_h, di, mb, am, sl, ss):
        cid = lax.axis_index("c")
        sid = lax.axis_index("s")
        wid = sid * _NC + cid
        r0 = sid * stride
        l0 = pltpu.async_copy(dst_h.at[wid], di, sl)
        pltpu.sync_copy(z_h.at[pl.ds(r0, span)], am.at[pl.ds(r0, span)])
        l0.wait()
        plsc.subcore_barrier()

        def slab(s, carry):
            base = wid * ew + s * cs
            l1 = pltpu.async_copy(mw_h.at[pl.ds(base, cs)], mb, sl)
            l1.wait()
            cps = []
            for k in range(_SSLAB):
                ch = s * _SSLAB + k
                cps.append(pltpu.async_copy(mb.at[pl.ds(k * _C, _C)],
                                            am.at[di.at[ch]], ss, add=True))
            for cp in cps:
                cp.wait()
            return carry

        lax.fori_loop(0, nslab, slab, 0)
        plsc.subcore_barrier()

        @pl.when(cid == 0)
        def _():
            pltpu.sync_copy(am.at[pl.ds(r0, span)], pm0_h.at[pl.ds(r0, span)])

        @pl.when(cid == 1)
        def _():
            pltpu.sync_copy(am.at[pl.ds(r0, span)], pm1_h.at[pl.ds(r0, span)])

    return scatter_k


def kernel(feats, pos, edge_index, x_t, t, T, W_in, b_in, We1, be1, We2, be2,
           Wx, bx, Wh1, bh1, Wh2, bh2, Wo1, bo1, Wo2, bo2):
    b, l_, f = feats.shape
    n = b * l_
    td = x_t.shape[-1]
    e = edge_index.shape[1]
    nl = We1.shape[0]
    hid = W_in.shape[1]
    inv_deg = float(n) / float(e)

    bn = 2000
    be_blk = 8000
    nb = n // bn

    t_norm = jnp.clip(t.astype(F32) / jnp.asarray(T).astype(F32), 0.0, 1.0)
    t_feat = jnp.broadcast_to(t_norm[:, None, None], (b, l_, 1))
    x_in = jnp.concatenate([feats, x_t, t_feat], axis=-1).reshape(n, -1)
    in_dim = x_in.shape[1]
    p4 = jnp.pad(pos.reshape(n, 3).astype(F32), ((0, 0), (0, _PW - 3)))
    e_a = e // 2
    sizes = [e_a, e - e_a]
    offs = [0, e_a]

    def _idx(rowi, sz, off, chunk):
        return edge_index[rowi, off:off + sz].reshape(
            _NW, (sz // _NW) // (_C * chunk) * chunk, _C)

    src_g = [_idx(0, sz, o, _GSLAB) for sz, o in zip(sizes, offs)]
    dst_g = [_idx(1, sz, o, _GSLAB) for sz, o in zip(sizes, offs)]
    dst_s = [_idx(1, sz, o, _SSLAB) for sz, o in zip(sizes, offs)]
    zmw = jnp.zeros((n, _W), F32)
    wo2p = jnp.pad(Wo2, ((0, 0), (0, _W - td)))
    bo2p = jnp.pad(bo2, (0, _W - td))

    row = lambda a: a.reshape(1, -1)

    wfull = lambda s: pl.BlockSpec(s, lambda i: (0, 0))
    nblk = lambda w: pl.BlockSpec((bn, w), lambda i: (i, 0))
    eblk = lambda w: pl.BlockSpec((be_blk, w), lambda i: (i, 0))

    h, tu, tv = pl.pallas_call(
        _init_body,
        grid=(nb,),
        in_specs=[
            nblk(in_dim), nblk(_PW), wfull((in_dim, hid)), wfull((1, hid)),
            wfull((hid, hid)), wfull((hid, hid)), wfull((1, hid)),
        ],
        out_specs=[nblk(hid), nblk(_W), nblk(_W)],
        out_shape=[
            jax.ShapeDtypeStruct((n, hid), F32),
            jax.ShapeDtypeStruct((n, _W), F32),
            jax.ShapeDtypeStruct((n, _W), F32),
        ],
    )(x_in, p4, W_in, row(b_in), We1[0, :hid], We1[0, hid:2 * hid],
      row(be1[0]))

    gather_ks = [_make_gather(sz) for sz in sizes]
    scatter_ks = [_make_scatter(n, sz) for sz in sizes]

    def edge_mlp(gs, gd, l, sz):
        return pl.pallas_call(
            _edge_body,
            grid=(sz // be_blk,),
            in_specs=[
                eblk(_W), eblk(_W),
                wfull((1, hid)), wfull((hid, hid)), wfull((1, hid)),
                wfull((1, hid)),
                pl.BlockSpec(memory_space=pltpu.SMEM),
            ],
            out_specs=eblk(_W),
            out_shape=jax.ShapeDtypeStruct((sz, _W), F32),
        )(gs, gd, row(We1[l, 2 * hid]), We2[l], row(be2[l]),
          row(Wx[l, :, 0]), bx[l].reshape(1, 1))

    p = p4
    for l in range(nl):
        gs_a, gd_a = gather_ks[0](tu, tv, src_g[0], dst_g[0])
        gs_b, gd_b = gather_ks[1](tu, tv, src_g[1], dst_g[1])
        mw_a = edge_mlp(gs_a, gd_a, l, sizes[0])
        mw_b = edge_mlp(gs_b, gd_b, l, sizes[1])
        pm0, pm1 = scatter_ks[0](mw_a, dst_s[0], zmw)
        pm2, pm3 = scatter_ks[1](mw_b, dst_s[1], zmw)

        if l + 1 < nl:
            la = l + 1
            h, p, tu, tv = pl.pallas_call(
                functools.partial(_node_body, inv_deg=inv_deg),
                grid=(nb,),
                in_specs=[
                    nblk(hid), nblk(_PW), nblk(_W), nblk(_W), nblk(_W),
                    nblk(_W),
                    wfull((hid, hid)), wfull((hid, hid)), wfull((1, hid)),
                    wfull((hid, hid)), wfull((1, hid)),
                    wfull((hid, hid)), wfull((hid, hid)), wfull((1, hid)),
                ],
                out_specs=[nblk(hid), nblk(_PW), nblk(_W), nblk(_W)],
                out_shape=[
                    jax.ShapeDtypeStruct((n, hid), F32),
                    jax.ShapeDtypeStruct((n, _PW), F32),
                    jax.ShapeDtypeStruct((n, _W), F32),
                    jax.ShapeDtypeStruct((n, _W), F32),
                ],
            )(h, p, pm0, pm1, pm2, pm3, Wh1[l, :hid], Wh1[l, hid:],
              row(bh1[l]), Wh2[l], row(bh2[l]), We1[la, :hid],
              We1[la, hid:2 * hid], row(be1[la]))
        else:
            pred = pl.pallas_call(
                functools.partial(_node_final_body, inv_deg=inv_deg),
                grid=(nb,),
                in_specs=[
                    nblk(hid), nblk(_W), nblk(_W), nblk(_W), nblk(_W),
                    wfull((hid, hid)), wfull((hid, hid)), wfull((1, hid)),
                    wfull((hid, hid)), wfull((1, hid)),
                    wfull((hid, hid)), wfull((1, hid)),
                    wfull((hid, _W)), wfull((1, _W)),
                ],
                out_specs=nblk(_W),
                out_shape=jax.ShapeDtypeStruct((n, _W), F32),
            )(h, pm0, pm1, pm2, pm3, Wh1[l, :hid], Wh1[l, hid:],
              row(bh1[l]), Wh2[l], row(bh2[l]), Wo1, row(bo1), wo2p,
              bo2p.reshape(1, -1))

    return pred[:, :td].reshape(b, l_, td)
